# initial kernel scaffold (unmeasured)
import jax
import jax.numpy as jnp
from jax import lax
from jax.experimental import pallas as pl
from jax.experimental.pallas import tpu as pltpu

N = 2048
D = 512
H = 1024
E = 128
WORLD = 32
E_LOC = E // WORLD
CAP = 128


def kernel(x, router_W, route_idx, expert_W):
    del router_W

    def body(x_ref, ridx_ref, ew_ref, out_ref, comm_ref, send_sems, recv_sems):
        me = lax.axis_index("i")

        bar = pltpu.get_barrier_semaphore()
        for d in range(1, WORLD):
            pl.semaphore_signal(
                bar, inc=1,
                device_id=((me + d) % WORLD,),
                device_id_type=pl.DeviceIdType.MESH,
            )
        pl.semaphore_wait(bar, WORLD - 1)

        r = ridx_ref[:, :]
        owner = r // E_LOC
        ohf = (owner == lax.broadcasted_iota(jnp.int32, (N, WORLD), 1)).astype(
            jnp.float32
        )
        low = (
            lax.broadcasted_iota(jnp.int32, (N, N), 1)
            < lax.broadcasted_iota(jnp.int32, (N, N), 0)
        ).astype(jnp.bfloat16)
        ranks = jnp.dot(low, ohf.astype(jnp.bfloat16),
                        preferred_element_type=jnp.float32)
        slot = jnp.sum(ranks * ohf, axis=1, keepdims=True)
        colf = owner.astype(jnp.float32) * CAP + slot

        mine = (owner == me).astype(jnp.float32)

        siota = lax.broadcasted_iota(jnp.float32, (CAP, N), 0)
        G = ((siota == slot.T) * mine.T).astype(jnp.bfloat16)

        xb = x_ref[:, :].astype(jnp.bfloat16)
        xc = jnp.dot(G, xb, preferred_element_type=jnp.float32).astype(
            jnp.bfloat16
        )
        eslot = jnp.dot(G, r.astype(jnp.bfloat16),
                        preferred_element_type=jnp.float32)

        y = jnp.zeros((CAP, H), jnp.float32)
        mef = me.astype(jnp.float32)
        for j in range(E_LOC):
            mj = (eslot == mef * E_LOC + j).astype(jnp.bfloat16)
            y = y + jnp.dot(xc * mj, ew_ref[j].astype(jnp.bfloat16),
                            preferred_element_type=jnp.float32)
        comm_ref[pl.ds(me * CAP, CAP), :] = y.astype(jnp.bfloat16)

        sends = []
        for d in range(1, WORLD):
            tgt = (me + d) % WORLD
            rdma = pltpu.make_async_remote_copy(
                src_ref=comm_ref.at[pl.ds(me * CAP, CAP)],
                dst_ref=comm_ref.at[pl.ds(me * CAP, CAP)],
                send_sem=send_sems.at[d - 1],
                recv_sem=recv_sems.at[d - 1],
                device_id=(tgt,),
                device_id_type=pl.DeviceIdType.MESH,
            )
            rdma.start()
            sends.append(rdma)

        out_ref[:, :] = jnp.zeros((N, H), jnp.float32)
        ci = lax.broadcasted_iota(jnp.float32, (N, CAP), 1)

        def accum(p):
            pf = p.astype(jnp.float32)
            Sp = (ci == (colf - pf * CAP)).astype(jnp.bfloat16)
            chunk = comm_ref[pl.ds(p * CAP, CAP), :]
            out_ref[:, :] = out_ref[:, :] + jnp.dot(
                Sp, chunk, preferred_element_type=jnp.float32
            )

        accum(me)
        for d in range(1, WORLD):
            p = (me + WORLD - d) % WORLD
            recv = pltpu.make_async_remote_copy(
                src_ref=comm_ref.at[pl.ds(p * CAP, CAP)],
                dst_ref=comm_ref.at[pl.ds(p * CAP, CAP)],
                send_sem=send_sems.at[d - 1],
                recv_sem=recv_sems.at[d - 1],
                device_id=(p,),
                device_id_type=pl.DeviceIdType.MESH,
            )
            recv.wait_recv()
            accum(p)

        for rdma in sends:
            rdma.wait_send()

    return pl.pallas_call(
        body,
        out_shape=jax.ShapeDtypeStruct((N, H), jnp.float32),
        in_specs=[
            pl.BlockSpec(memory_space=pltpu.VMEM),
            pl.BlockSpec(memory_space=pltpu.VMEM),
            pl.BlockSpec(memory_space=pltpu.VMEM),
        ],
        out_specs=pl.BlockSpec(memory_space=pltpu.VMEM),
        scratch_shapes=[
            pltpu.VMEM((WORLD * CAP, H), jnp.bfloat16),
            pltpu.SemaphoreType.DMA((WORLD - 1,)),
            pltpu.SemaphoreType.DMA((WORLD - 1,)),
        ],
        compiler_params=pltpu.CompilerParams(collective_id=0),
    )(x, route_idx, expert_W)


# baseline (device time: 153139 ns/iter reference)
import jax
import jax.numpy as jnp
from jax import lax
from jax.experimental import pallas as pl
from jax.experimental.pallas import tpu as pltpu

N = 2048
D = 512
H = 1024
E = 128
WORLD = 32
E_LOC = E // WORLD
CAP = 128


def kernel(x, router_W, route_idx, expert_W):
    del router_W

    def body(x_ref, ridx_ref, ew_ref, out_ref, comm_ref, send_sems, recv_sems):
        me = lax.axis_index("i")

        bar = pltpu.get_barrier_semaphore()
        for d in range(1, WORLD):
            pl.semaphore_signal(
                bar, inc=1,
                device_id=((me + d) % WORLD,),
                device_id_type=pl.DeviceIdType.MESH,
            )
        pl.semaphore_wait(bar, WORLD - 1)

        r = ridx_ref[:, :]
        owner = r // E_LOC
        ohf = (owner == lax.broadcasted_iota(jnp.int32, (N, WORLD), 1)).astype(
            jnp.float32
        )
        low = (
            lax.broadcasted_iota(jnp.int32, (N, N), 1)
            < lax.broadcasted_iota(jnp.int32, (N, N), 0)
        ).astype(jnp.bfloat16)
        ranks = jnp.dot(low, ohf.astype(jnp.bfloat16),
                        preferred_element_type=jnp.float32)
        slot = jnp.sum(ranks * ohf, axis=1, keepdims=True)
        colf = owner.astype(jnp.float32) * CAP + slot

        mine = (owner == me).astype(jnp.float32)

        siota = lax.broadcasted_iota(jnp.int32, (CAP, N), 0).astype(jnp.float32)
        G = ((siota == slot.T) * mine.T).astype(jnp.bfloat16)

        xb = x_ref[:, :].astype(jnp.bfloat16)
        xc = jnp.dot(G, xb, preferred_element_type=jnp.float32).astype(
            jnp.bfloat16
        )
        eslot = jnp.dot(G, r.astype(jnp.bfloat16),
                        preferred_element_type=jnp.float32)

        y = jnp.zeros((CAP, H), jnp.float32)
        mef = me.astype(jnp.float32)
        for j in range(E_LOC):
            mj = (eslot == mef * E_LOC + j).astype(jnp.bfloat16)
            y = y + jnp.dot(xc * mj, ew_ref[j].astype(jnp.bfloat16),
                            preferred_element_type=jnp.float32)
        comm_ref[pl.ds(me * CAP, CAP), :] = y.astype(jnp.bfloat16)

        sends = []
        for d in range(1, WORLD):
            tgt = (me + d) % WORLD
            rdma = pltpu.make_async_remote_copy(
                src_ref=comm_ref.at[pl.ds(me * CAP, CAP)],
                dst_ref=comm_ref.at[pl.ds(me * CAP, CAP)],
                send_sem=send_sems.at[d - 1],
                recv_sem=recv_sems.at[d - 1],
                device_id=(tgt,),
                device_id_type=pl.DeviceIdType.MESH,
            )
            rdma.start()
            sends.append(rdma)

        out_ref[:, :] = jnp.zeros((N, H), jnp.float32)
        ci = lax.broadcasted_iota(jnp.int32, (N, CAP), 1).astype(jnp.float32)

        def accum(p):
            pf = p.astype(jnp.float32)
            Sp = (ci == (colf - pf * CAP)).astype(jnp.bfloat16)
            chunk = comm_ref[pl.ds(p * CAP, CAP), :]
            out_ref[:, :] = out_ref[:, :] + jnp.dot(
                Sp, chunk, preferred_element_type=jnp.float32
            )

        accum(me)
        for d in range(1, WORLD):
            p = (me + WORLD - d) % WORLD
            recv = pltpu.make_async_remote_copy(
                src_ref=comm_ref.at[pl.ds(p * CAP, CAP)],
                dst_ref=comm_ref.at[pl.ds(p * CAP, CAP)],
                send_sem=send_sems.at[d - 1],
                recv_sem=recv_sems.at[d - 1],
                device_id=(p,),
                device_id_type=pl.DeviceIdType.MESH,
            )
            recv.wait_recv()
            accum(p)

        for rdma in sends:
            rdma.wait_send()

    return pl.pallas_call(
        body,
        out_shape=jax.ShapeDtypeStruct((N, H), jnp.float32),
        in_specs=[
            pl.BlockSpec(memory_space=pltpu.VMEM),
            pl.BlockSpec(memory_space=pltpu.VMEM),
            pl.BlockSpec(memory_space=pltpu.VMEM),
        ],
        out_specs=pl.BlockSpec(memory_space=pltpu.VMEM),
        scratch_shapes=[
            pltpu.VMEM((WORLD * CAP, H), jnp.bfloat16),
            pltpu.SemaphoreType.DMA((WORLD - 1,)),
            pltpu.SemaphoreType.DMA((WORLD - 1,)),
        ],
        compiler_params=pltpu.CompilerParams(collective_id=0),
    )(x, route_idx, expert_W)
